# Initial kernel scaffold; baseline (speedup 1.0000x reference)
import functools

import jax
import jax.numpy as jnp
from jax import lax
from jax.experimental import pallas as pl
from jax.experimental.pallas import tpu as pltpu
from jax.experimental.pallas import tpu_sc as plsc

N = 10000
E = 320000
D = 128
NC = 2
NS = 16
NW = NC * NS
EPW = E // NW
CHUNK = 80
ITERS = EPW // CHUNK
RPS = N // NS

_EPS_BN = 1e-5
_EPS_NORM = 1e-12

_mesh = plsc.VectorSubcoreMesh(core_axis_name="c", subcore_axis_name="s")


@functools.partial(
    pl.kernel,
    out_type=[
        jax.ShapeDtypeStruct((NC, N, D), jnp.float32),
        jax.ShapeDtypeStruct((NC, N, 1), jnp.float32),
    ],
    mesh=_mesh,
    scratch_types=[
        pltpu.VMEM((CHUNK,), jnp.int32),
        pltpu.VMEM((CHUNK,), jnp.int32),
        pltpu.VMEM((CHUNK, D), jnp.float32),
        pltpu.VMEM((CHUNK, 1), jnp.float32),
        pltpu.VMEM_SHARED((N, D), jnp.float32),
        pltpu.VMEM_SHARED((N, 1), jnp.float32),
        pltpu.SemaphoreType.DMA,
    ],
)
def _agg_counts(table, src, dst, zfeat, zcnt, ones, sums_out, cnt_out,
                sidx, didx, rows, ones_v, accum, cacc, sem):
    c = lax.axis_index("c")
    s = lax.axis_index("s")
    w = s * NC + c
    base0 = s * RPS
    pltpu.sync_copy(zfeat.at[pl.ds(base0, RPS)], accum.at[pl.ds(base0, RPS)])
    pltpu.sync_copy(zcnt.at[pl.ds(base0, RPS)], cacc.at[pl.ds(base0, RPS)])
    pltpu.sync_copy(ones, ones_v)
    plsc.subcore_barrier()

    wbase = w * EPW

    def step(j, carry):
        base = wbase + j * CHUNK
        pltpu.sync_copy(src.at[pl.ds(base, CHUNK)], sidx)
        pltpu.sync_copy(dst.at[pl.ds(base, CHUNK)], didx)
        pltpu.async_copy(table.at[sidx], rows, sem).wait()
        pltpu.sync_copy(rows, accum.at[didx], add=True)
        pltpu.sync_copy(ones_v, cacc.at[didx], add=True)
        return carry

    lax.fori_loop(0, ITERS, step, 0)
    plsc.subcore_barrier()
    pltpu.sync_copy(accum.at[pl.ds(base0, RPS)],
                    sums_out.at[c, pl.ds(base0, RPS)])
    pltpu.sync_copy(cacc.at[pl.ds(base0, RPS)],
                    cnt_out.at[c, pl.ds(base0, RPS)])


@functools.partial(
    pl.kernel,
    out_type=jax.ShapeDtypeStruct((NC, N, D), jnp.float32),
    mesh=_mesh,
    scratch_types=[
        pltpu.VMEM((CHUNK,), jnp.int32),
        pltpu.VMEM((CHUNK,), jnp.int32),
        pltpu.VMEM((CHUNK, D), jnp.float32),
        pltpu.VMEM_SHARED((N, D), jnp.float32),
        pltpu.SemaphoreType.DMA,
    ],
)
def _agg(table, src, dst, zfeat, sums_out, sidx, didx, rows, accum, sem):
    c = lax.axis_index("c")
    s = lax.axis_index("s")
    w = s * NC + c
    base0 = s * RPS
    pltpu.sync_copy(zfeat.at[pl.ds(base0, RPS)], accum.at[pl.ds(base0, RPS)])
    plsc.subcore_barrier()

    wbase = w * EPW

    def step(j, carry):
        base = wbase + j * CHUNK
        pltpu.sync_copy(src.at[pl.ds(base, CHUNK)], sidx)
        pltpu.sync_copy(dst.at[pl.ds(base, CHUNK)], didx)
        pltpu.async_copy(table.at[sidx], rows, sem).wait()
        pltpu.sync_copy(rows, accum.at[didx], add=True)
        return carry

    lax.fori_loop(0, ITERS, step, 0)
    plsc.subcore_barrier()
    pltpu.sync_copy(accum.at[pl.ds(base0, RPS)],
                    sums_out.at[c, pl.ds(base0, RPS)])


_R = 1000


def _dense1_body(sp_ref, cp_ref, x_ref, wl_ref, bl_ref, wr_ref, g_ref, b_ref,
                 o_ref):
    ssum = sp_ref[0] + sp_ref[1]
    cnt = cp_ref[0] + cp_ref[1]
    mean = ssum / jnp.maximum(cnt, 1.0)
    out = (jnp.dot(mean, wl_ref[...], preferred_element_type=jnp.float32)
           + jnp.dot(x_ref[...], wr_ref[...], preferred_element_type=jnp.float32)
           + bl_ref[...])
    nrm = jnp.sqrt(jnp.sum(out * out, axis=1, keepdims=True))
    out = out / jnp.maximum(nrm, _EPS_NORM)
    out = jnp.maximum(out, 0.0)
    o_ref[...] = g_ref[...] * out * (1.0 / jnp.sqrt(1.0 + _EPS_BN)) + b_ref[...]


def _dense2_body(sp_ref, cp_ref, h_ref, wl_ref, bl_ref, wr_ref, wfc_ref,
                 bfc_ref, o_ref):
    ssum = sp_ref[0] + sp_ref[1]
    cnt = cp_ref[0] + cp_ref[1]
    mean = ssum / jnp.maximum(cnt, 1.0)
    out = (jnp.dot(mean, wl_ref[...], preferred_element_type=jnp.float32)
           + jnp.dot(h_ref[...], wr_ref[...], preferred_element_type=jnp.float32)
           + bl_ref[...])
    nrm = jnp.sqrt(jnp.sum(out * out, axis=1, keepdims=True))
    out = out / jnp.maximum(nrm, _EPS_NORM)
    o_ref[...] = (jnp.sum(out * wfc_ref[...], axis=1, keepdims=True)
                  + bfc_ref[...])


def _row_specs():
    return [
        pl.BlockSpec((NC, _R, D), lambda i: (0, i, 0)),
        pl.BlockSpec((NC, _R, 1), lambda i: (0, i, 0)),
        pl.BlockSpec((_R, D), lambda i: (i, 0)),
    ]


def _full2d(shape):
    return pl.BlockSpec(shape, lambda i: (0, 0))


def _dense1(sp, cp, x, wl, bl, wr, g, b):
    return pl.pallas_call(
        _dense1_body,
        grid=(N // _R,),
        in_specs=_row_specs() + [
            _full2d((D, D)), _full2d((1, D)), _full2d((D, D)),
            _full2d((1, D)), _full2d((1, D)),
        ],
        out_specs=pl.BlockSpec((_R, D), lambda i: (i, 0)),
        out_shape=jax.ShapeDtypeStruct((N, D), jnp.float32),
    )(sp, cp, x, wl, bl, wr, g, b)


def _dense2(sp, cp, h, wl, bl, wr, wfc, bfc):
    return pl.pallas_call(
        _dense2_body,
        grid=(N // _R,),
        in_specs=_row_specs() + [
            _full2d((D, D)), _full2d((1, D)), _full2d((D, D)),
            _full2d((1, D)), _full2d((1, 1)),
        ],
        out_specs=pl.BlockSpec((_R, 1), lambda i: (i, 0)),
        out_shape=jax.ShapeDtypeStruct((N, 1), jnp.float32),
    )(sp, cp, h, wl, bl, wr, wfc, bfc)


def kernel(x, edge_index, W1l, b1l, W1r, gamma, beta, W2l, b2l, W2r, Wfc, bfc):
    src = edge_index[0]
    dst = edge_index[1]
    zfeat = jnp.zeros((N, D), jnp.float32)
    zcnt = jnp.zeros((N, 1), jnp.float32)
    ones = jnp.ones((CHUNK, 1), jnp.float32)

    sums1, cnts = _agg_counts(x, src, dst, zfeat, zcnt, ones)
    h = _dense1(sums1, cnts, x, W1l, b1l.reshape(1, D), W1r,
                gamma.reshape(1, D), beta.reshape(1, D))
    sums2 = _agg(h, src, dst, zfeat)
    out = _dense2(sums2, cnts, h, W2l, b2l.reshape(1, D), W2r,
                  Wfc.reshape(1, D), bfc.reshape(1, 1))
    return out.reshape(N)


# speedup vs baseline: 4.7745x; 4.7745x over previous
import functools

import jax
import jax.numpy as jnp
from jax import lax
from jax.experimental import pallas as pl
from jax.experimental.pallas import tpu as pltpu
from jax.experimental.pallas import tpu_sc as plsc

N = 10000
E = 320000
D = 128
NC = 2
NS = 16
NW = NC * NS
EPW = E // NW
CHUNK = 80
ITERS = EPW // CHUNK
RPS = 624
TAIL = N - NS * RPS
TAIL_BASE = NS * RPS

_EPS_BN = 1e-5
_EPS_NORM = 1e-12

_mesh = plsc.VectorSubcoreMesh(core_axis_name="c", subcore_axis_name="s")


def _zero_stripe(zfeat, accum, s):
    base0 = s * RPS
    pltpu.sync_copy(zfeat.at[pl.ds(base0, RPS)], accum.at[pl.ds(base0, RPS)])

    @pl.when(s == NS - 1)
    def _tail():
        pltpu.sync_copy(zfeat.at[pl.ds(TAIL_BASE, TAIL)],
                        accum.at[pl.ds(TAIL_BASE, TAIL)])


def _write_stripe(accum, out, c, s):
    base0 = s * RPS
    pltpu.sync_copy(accum.at[pl.ds(base0, RPS)],
                    out.at[c, pl.ds(base0, RPS)])

    @pl.when(s == NS - 1)
    def _tail():
        pltpu.sync_copy(accum.at[pl.ds(TAIL_BASE, TAIL)],
                        out.at[c, pl.ds(TAIL_BASE, TAIL)])


@functools.partial(
    pl.kernel,
    out_type=[
        jax.ShapeDtypeStruct((NC, N, D), jnp.float32),
        jax.ShapeDtypeStruct((NC, N, D), jnp.float32),
    ],
    mesh=_mesh,
    scratch_types=[
        pltpu.VMEM((CHUNK,), jnp.int32),
        pltpu.VMEM((CHUNK,), jnp.int32),
        pltpu.VMEM((CHUNK, D), jnp.float32),
        pltpu.VMEM((CHUNK, D), jnp.float32),
        pltpu.VMEM_SHARED((N, D), jnp.float32),
        pltpu.SemaphoreType.DMA,
    ],
)
def _agg_counts(table, src, dst, zfeat, ones, sums_out, cnt_out,
                sidx, didx, rows, ones_v, accum, sem):
    c = lax.axis_index("c")
    s = lax.axis_index("s")
    w = s * NC + c
    wbase = w * EPW

    _zero_stripe(zfeat, accum, s)
    pltpu.sync_copy(ones, ones_v)
    plsc.subcore_barrier()

    def cstep(j, carry):
        pltpu.sync_copy(dst.at[pl.ds(wbase + j * CHUNK, CHUNK)], didx)
        pltpu.sync_copy(ones_v, accum.at[didx], add=True)
        return carry

    lax.fori_loop(0, ITERS, cstep, 0)
    plsc.subcore_barrier()
    _write_stripe(accum, cnt_out, c, s)

    _zero_stripe(zfeat, accum, s)
    plsc.subcore_barrier()

    def step(j, carry):
        base = wbase + j * CHUNK
        pltpu.sync_copy(src.at[pl.ds(base, CHUNK)], sidx)
        pltpu.sync_copy(dst.at[pl.ds(base, CHUNK)], didx)
        pltpu.async_copy(table.at[sidx], rows, sem).wait()
        pltpu.sync_copy(rows, accum.at[didx], add=True)
        return carry

    lax.fori_loop(0, ITERS, step, 0)
    plsc.subcore_barrier()
    _write_stripe(accum, sums_out, c, s)


@functools.partial(
    pl.kernel,
    out_type=jax.ShapeDtypeStruct((NC, N, D), jnp.float32),
    mesh=_mesh,
    scratch_types=[
        pltpu.VMEM((CHUNK,), jnp.int32),
        pltpu.VMEM((CHUNK,), jnp.int32),
        pltpu.VMEM((CHUNK, D), jnp.float32),
        pltpu.VMEM_SHARED((N, D), jnp.float32),
        pltpu.SemaphoreType.DMA,
    ],
)
def _agg(table, src, dst, zfeat, sums_out, sidx, didx, rows, accum, sem):
    c = lax.axis_index("c")
    s = lax.axis_index("s")
    w = s * NC + c
    wbase = w * EPW
    _zero_stripe(zfeat, accum, s)
    plsc.subcore_barrier()

    def step(j, carry):
        base = wbase + j * CHUNK
        pltpu.sync_copy(src.at[pl.ds(base, CHUNK)], sidx)
        pltpu.sync_copy(dst.at[pl.ds(base, CHUNK)], didx)
        pltpu.async_copy(table.at[sidx], rows, sem).wait()
        pltpu.sync_copy(rows, accum.at[didx], add=True)
        return carry

    lax.fori_loop(0, ITERS, step, 0)
    plsc.subcore_barrier()
    _write_stripe(accum, sums_out, c, s)


_R = 1000


def _dense1_body(sp_ref, cp_ref, x_ref, wl_ref, bl_ref, wr_ref, g_ref, b_ref,
                 o_ref):
    ssum = sp_ref[0] + sp_ref[1]
    cnt = cp_ref[0][:, 0:1] + cp_ref[1][:, 0:1]
    mean = ssum / jnp.maximum(cnt, 1.0)
    out = (jnp.dot(mean, wl_ref[...], preferred_element_type=jnp.float32)
           + jnp.dot(x_ref[...], wr_ref[...], preferred_element_type=jnp.float32)
           + bl_ref[...])
    nrm = jnp.sqrt(jnp.sum(out * out, axis=1, keepdims=True))
    out = out / jnp.maximum(nrm, _EPS_NORM)
    out = jnp.maximum(out, 0.0)
    o_ref[...] = g_ref[...] * out * (1.0 / jnp.sqrt(1.0 + _EPS_BN)) + b_ref[...]


def _dense2_body(sp_ref, cp_ref, h_ref, wl_ref, bl_ref, wr_ref, wfc_ref,
                 bfc_ref, o_ref):
    ssum = sp_ref[0] + sp_ref[1]
    cnt = cp_ref[0][:, 0:1] + cp_ref[1][:, 0:1]
    mean = ssum / jnp.maximum(cnt, 1.0)
    out = (jnp.dot(mean, wl_ref[...], preferred_element_type=jnp.float32)
           + jnp.dot(h_ref[...], wr_ref[...], preferred_element_type=jnp.float32)
           + bl_ref[...])
    nrm = jnp.sqrt(jnp.sum(out * out, axis=1, keepdims=True))
    out = out / jnp.maximum(nrm, _EPS_NORM)
    o_ref[...] = (jnp.sum(out * wfc_ref[...], axis=1, keepdims=True)
                  + bfc_ref[...])


def _row_specs():
    return [
        pl.BlockSpec((NC, _R, D), lambda i: (0, i, 0)),
        pl.BlockSpec((NC, _R, D), lambda i: (0, i, 0)),
        pl.BlockSpec((_R, D), lambda i: (i, 0)),
    ]


def _full2d(shape):
    return pl.BlockSpec(shape, lambda i: (0, 0))


def _dense1(sp, cp, x, wl, bl, wr, g, b):
    return pl.pallas_call(
        _dense1_body,
        grid=(N // _R,),
        in_specs=_row_specs() + [
            _full2d((D, D)), _full2d((1, D)), _full2d((D, D)),
            _full2d((1, D)), _full2d((1, D)),
        ],
        out_specs=pl.BlockSpec((_R, D), lambda i: (i, 0)),
        out_shape=jax.ShapeDtypeStruct((N, D), jnp.float32),
    )(sp, cp, x, wl, bl, wr, g, b)


def _dense2(sp, cp, h, wl, bl, wr, wfc, bfc):
    return pl.pallas_call(
        _dense2_body,
        grid=(N // _R,),
        in_specs=_row_specs() + [
            _full2d((D, D)), _full2d((1, D)), _full2d((D, D)),
            _full2d((1, D)), _full2d((1, 1)),
        ],
        out_specs=pl.BlockSpec((_R, 1), lambda i: (i, 0)),
        out_shape=jax.ShapeDtypeStruct((N, 1), jnp.float32),
    )(sp, cp, h, wl, bl, wr, wfc, bfc)


def kernel(x, edge_index, W1l, b1l, W1r, gamma, beta, W2l, b2l, W2r, Wfc, bfc):
    src = edge_index[0]
    dst = edge_index[1]
    zfeat = jnp.zeros((N, D), jnp.float32)
    ones = jnp.ones((CHUNK, D), jnp.float32)

    sums1, cnts = _agg_counts(x, src, dst, zfeat, ones)
    h = _dense1(sums1, cnts, x, W1l, b1l.reshape(1, D), W1r,
                gamma.reshape(1, D), beta.reshape(1, D))
    sums2 = _agg(h, src, dst, zfeat)
    out = _dense2(sums2, cnts, h, W2l, b2l.reshape(1, D), W2r,
                  Wfc.reshape(1, D), bfc.reshape(1, 1))
    return out.reshape(N)
